# baseline (speedup 1.0000x reference)
import jax
import jax.numpy as jnp
from jax.experimental import pallas as pl

B, D, H, W = 4, 192, 224, 224
HW = H * W
LANES = 128
ROWS = HW // LANES
NJ = 49
R = ROWS // NJ
EPS = 1e-8


def _loss_block_kernel(s_ref, t_ref, ox_ref, m_ref, c_ref, lsum_ref, cnt_ref):
    j = pl.program_id(1)

    @pl.when(j == 0)
    def _init():
        lsum_ref[...] = jnp.zeros_like(lsum_ref)
        cnt_ref[...] = jnp.zeros_like(cnt_ref)

    s = s_ref[0]
    t = t_ref[0] - c_ref[...]
    st = jnp.sum(s * t, axis=0)
    ss = jnp.sum(s * s, axis=0)
    tt = jnp.sum(t * t, axis=0)
    s_n = jnp.maximum(jnp.sqrt(ss), EPS)
    t_n = jnp.maximum(jnp.sqrt(tt), EPS)
    loss = 1.0 - st / (s_n * t_n)
    valid = (ox_ref[0] != 0.0) & (m_ref[0] == 0.0)
    vf = valid.astype(jnp.float32)
    lsum_ref[...] += jnp.sum(loss * vf, axis=0, keepdims=True).reshape(1, 1, LANES)
    cnt_ref[...] += jnp.sum(vf, axis=0, keepdims=True).reshape(1, 1, LANES)


def _finalize_kernel(lsum_ref, cnt_ref, out_ref):
    ls = jnp.sum(lsum_ref[:, 0, :], axis=1, keepdims=True)
    cn = jnp.sum(cnt_ref[:, 0, :], axis=1, keepdims=True)
    per = ls / jnp.clip(cn, 1.0, None)
    hv = (cn > 0.0).astype(jnp.float32)
    num = jnp.sum(per * hv, keepdims=True).reshape(1, 1)
    den = jnp.maximum(jnp.sum(hv, keepdims=True).reshape(1, 1), 1.0)
    total = jnp.sum(cn, keepdims=True).reshape(1, 1)
    out_ref[...] = jnp.where(total == 0.0, 0.0, num / den)


def kernel(student_feats, teacher_feats, mask, original_x, center):
    s = student_feats.reshape(B, D, ROWS, LANES)
    t = teacher_feats.reshape(B, D, ROWS, LANES)
    ox = original_x.reshape(B, ROWS, LANES)
    m = mask.reshape(B, ROWS, LANES).astype(jnp.float32)
    c = center.reshape(D, 1, 1)

    lsum, cnt = pl.pallas_call(
        _loss_block_kernel,
        grid=(B, NJ),
        in_specs=[
            pl.BlockSpec((1, D, R, LANES), lambda b, j: (b, 0, j, 0)),
            pl.BlockSpec((1, D, R, LANES), lambda b, j: (b, 0, j, 0)),
            pl.BlockSpec((1, R, LANES), lambda b, j: (b, j, 0)),
            pl.BlockSpec((1, R, LANES), lambda b, j: (b, j, 0)),
            pl.BlockSpec((D, 1, 1), lambda b, j: (0, 0, 0)),
        ],
        out_specs=[
            pl.BlockSpec((1, 1, LANES), lambda b, j: (b, 0, 0)),
            pl.BlockSpec((1, 1, LANES), lambda b, j: (b, 0, 0)),
        ],
        out_shape=[
            jax.ShapeDtypeStruct((B, 1, LANES), jnp.float32),
            jax.ShapeDtypeStruct((B, 1, LANES), jnp.float32),
        ],
    )(s, t, ox, m, c)

    out = pl.pallas_call(
        _finalize_kernel,
        out_shape=jax.ShapeDtypeStruct((1, 1), jnp.float32),
    )(lsum, cnt)
    return out[0, 0]


# speedup vs baseline: 1.3473x; 1.3473x over previous
import jax
import jax.numpy as jnp
from jax.experimental import pallas as pl
from jax.experimental.pallas import tpu as pltpu

B, D, H, W = 4, 192, 224, 224
HW = H * W
DC = 32
ND = D // DC
EPS = 1e-8


def _loss_block_kernel(s_ref, t_ref, ox_ref, m_ref, c_ref,
                       lv_ref, vf_ref, st_ref, ss_ref, tt_ref):
    k = pl.program_id(1)

    @pl.when(k == 0)
    def _init():
        st_ref[...] = jnp.zeros_like(st_ref)
        ss_ref[...] = jnp.zeros_like(ss_ref)
        tt_ref[...] = jnp.zeros_like(tt_ref)

    s = s_ref[0]
    t = t_ref[0] - c_ref[0]
    st_ref[...] += jnp.sum(s * t, axis=0, keepdims=True)
    ss_ref[...] += jnp.sum(s * s, axis=0, keepdims=True)
    tt_ref[...] += jnp.sum(t * t, axis=0, keepdims=True)

    @pl.when(k == ND - 1)
    def _final():
        s_n = jnp.maximum(jnp.sqrt(ss_ref[...]), EPS)
        t_n = jnp.maximum(jnp.sqrt(tt_ref[...]), EPS)
        loss = 1.0 - st_ref[...] / (s_n * t_n)
        valid = (ox_ref[0] != 0.0) & (m_ref[0] == 0.0)
        vf = valid.astype(jnp.float32)
        lv_ref[0] = loss * vf
        vf_ref[0] = vf


def _finalize_kernel(lv_ref, vf_ref, out_ref):
    ls = jnp.sum(lv_ref[:, 0, :], axis=1, keepdims=True)
    cn = jnp.sum(vf_ref[:, 0, :], axis=1, keepdims=True)
    per = ls / jnp.clip(cn, 1.0, None)
    hv = (cn > 0.0).astype(jnp.float32)
    num = jnp.sum(per * hv, keepdims=True).reshape(1, 1)
    den = jnp.maximum(jnp.sum(hv, keepdims=True).reshape(1, 1), 1.0)
    total = jnp.sum(cn, keepdims=True).reshape(1, 1)
    out_ref[...] = jnp.where(total == 0.0, 0.0, num / den)


def kernel(student_feats, teacher_feats, mask, original_x, center):
    s = student_feats.reshape(B, D, HW)
    t = teacher_feats.reshape(B, D, HW)
    ox = original_x.reshape(B, 1, HW)
    m = mask.reshape(B, 1, HW).astype(jnp.float32)
    c = center.reshape(ND, DC, 1)

    lv, vf = pl.pallas_call(
        _loss_block_kernel,
        grid=(B, ND),
        in_specs=[
            pl.BlockSpec((1, DC, HW), lambda b, k: (b, k, 0)),
            pl.BlockSpec((1, DC, HW), lambda b, k: (b, k, 0)),
            pl.BlockSpec((1, 1, HW), lambda b, k: (b, 0, 0)),
            pl.BlockSpec((1, 1, HW), lambda b, k: (b, 0, 0)),
            pl.BlockSpec((1, DC, 1), lambda b, k: (k, 0, 0)),
        ],
        out_specs=[
            pl.BlockSpec((1, 1, HW), lambda b, k: (b, 0, 0)),
            pl.BlockSpec((1, 1, HW), lambda b, k: (b, 0, 0)),
        ],
        out_shape=[
            jax.ShapeDtypeStruct((B, 1, HW), jnp.float32),
            jax.ShapeDtypeStruct((B, 1, HW), jnp.float32),
        ],
        scratch_shapes=[
            pltpu.VMEM((1, HW), jnp.float32),
            pltpu.VMEM((1, HW), jnp.float32),
            pltpu.VMEM((1, HW), jnp.float32),
        ],
    )(s, t, ox, m, c)

    out = pl.pallas_call(
        _finalize_kernel,
        out_shape=jax.ShapeDtypeStruct((1, 1), jnp.float32),
    )(lv, vf)
    return out[0, 0]


# speedup vs baseline: 4.6045x; 3.4176x over previous
import jax
import jax.numpy as jnp
from jax.experimental import pallas as pl
from jax.experimental.pallas import tpu as pltpu

B, D, H, W = 4, 192, 224, 224
DC = 32
ND = D // DC
NSUB = DC // 8
EPS = 1e-8


def _loss_block_kernel(s_ref, t_ref, ox_ref, m_ref, c_ref,
                       lv_ref, vf_ref, st_ref, ss_ref, tt_ref):
    k = pl.program_id(1)

    @pl.when(k == 0)
    def _init():
        st_ref[...] = jnp.zeros_like(st_ref)
        ss_ref[...] = jnp.zeros_like(ss_ref)
        tt_ref[...] = jnp.zeros_like(tt_ref)

    s = s_ref[0]
    t = t_ref[0] - c_ref[0]
    pst = s * t
    pss = s * s
    ptt = t * t
    st_acc = pst[0:8]
    ss_acc = pss[0:8]
    tt_acc = ptt[0:8]
    for g in range(1, NSUB):
        st_acc = st_acc + pst[8 * g:8 * (g + 1)]
        ss_acc = ss_acc + pss[8 * g:8 * (g + 1)]
        tt_acc = tt_acc + ptt[8 * g:8 * (g + 1)]
    st_ref[...] += st_acc
    ss_ref[...] += ss_acc
    tt_ref[...] += tt_acc

    @pl.when(k == ND - 1)
    def _final():
        st = jnp.sum(st_ref[...], axis=0)
        ss = jnp.sum(ss_ref[...], axis=0)
        tt = jnp.sum(tt_ref[...], axis=0)
        s_n = jnp.maximum(jnp.sqrt(ss), EPS)
        t_n = jnp.maximum(jnp.sqrt(tt), EPS)
        loss = 1.0 - st / (s_n * t_n)
        valid = (ox_ref[0, 0] != 0.0) & (m_ref[0] == 0.0)
        vf = valid.astype(jnp.float32)
        lv_ref[0] = loss * vf
        vf_ref[0] = vf


def _finalize_kernel(lv_ref, vf_ref, out_ref):
    ls = jnp.sum(jnp.sum(lv_ref[...], axis=2), axis=1, keepdims=True)
    cn = jnp.sum(jnp.sum(vf_ref[...], axis=2), axis=1, keepdims=True)
    per = ls / jnp.clip(cn, 1.0, None)
    hv = (cn > 0.0).astype(jnp.float32)
    num = jnp.sum(per * hv, keepdims=True).reshape(1, 1)
    den = jnp.maximum(jnp.sum(hv, keepdims=True).reshape(1, 1), 1.0)
    total = jnp.sum(cn, keepdims=True).reshape(1, 1)
    out_ref[...] = jnp.where(total == 0.0, 0.0, num / den)


def kernel(student_feats, teacher_feats, mask, original_x, center):
    m = mask.astype(jnp.float32)
    c = center.reshape(ND, DC, 1, 1)

    lv, vf = pl.pallas_call(
        _loss_block_kernel,
        grid=(B, ND),
        in_specs=[
            pl.BlockSpec((1, DC, H, W), lambda b, k: (b, k, 0, 0)),
            pl.BlockSpec((1, DC, H, W), lambda b, k: (b, k, 0, 0)),
            pl.BlockSpec((1, 1, H, W), lambda b, k: (b, 0, 0, 0)),
            pl.BlockSpec((1, H, W), lambda b, k: (b, 0, 0)),
            pl.BlockSpec((1, DC, 1, 1), lambda b, k: (k, 0, 0, 0)),
        ],
        out_specs=[
            pl.BlockSpec((1, H, W), lambda b, k: (b, 0, 0)),
            pl.BlockSpec((1, H, W), lambda b, k: (b, 0, 0)),
        ],
        out_shape=[
            jax.ShapeDtypeStruct((B, H, W), jnp.float32),
            jax.ShapeDtypeStruct((B, H, W), jnp.float32),
        ],
        scratch_shapes=[
            pltpu.VMEM((8, H, W), jnp.float32),
            pltpu.VMEM((8, H, W), jnp.float32),
            pltpu.VMEM((8, H, W), jnp.float32),
        ],
    )(student_feats, teacher_feats, original_x, m, c)

    out = pl.pallas_call(
        _finalize_kernel,
        out_shape=jax.ShapeDtypeStruct((1, 1), jnp.float32),
    )(lv, vf)
    return out[0, 0]
